# Initial kernel scaffold; baseline (speedup 1.0000x reference)
import jax
import jax.numpy as jnp
from jax.experimental import pallas as pl


def kernel(x, t, W, b):
    raise NotImplementedError("write your pallas kernel here")


# speedup vs baseline: 17.6576x; 17.6576x over previous
import functools

import jax
import jax.numpy as jnp
from jax import lax
from jax.experimental import pallas as pl
from jax.experimental.pallas import tpu as pltpu
from jax.experimental.pallas import tpu_sc as plsc

B, T, D = 4, 256, 128
NPAD = 2568
NROWS = B * T * T
NC, NS = 2, 16
NW = NC * NS
ROWS_PER_W = NROWS // NW
CHUNK = 128
NCHUNK = ROWS_PER_W // CHUNK


def _tc_body(t_ref, tT_ref, x_ref, W_ref, b_ref, p_ref, ytab_ref, gidx_ref):
    bi = pl.program_id(0)
    delta = p_ref[0, 0, 0]
    tt0 = p_ref[0, 0, 1]
    ext = p_ref[0, 0, 2]

    tr = t_ref[0]
    s = lax.broadcasted_iota(jnp.int32, (NPAD, 1), 0).astype(jnp.float32) * delta

    cmp = (tr <= s).astype(jnp.int32)
    cnt = jnp.sum(cmp, axis=1, keepdims=True)
    cp = jnp.clip(cnt, 1, T - 1)
    kk = lax.broadcasted_iota(jnp.int32, (NPAD, T), 1)
    m1 = (kk == cp - 1).astype(jnp.float32)
    m2 = (kk == cp).astype(jnp.float32)
    t_prev = jnp.sum(m1 * tr, axis=1, keepdims=True)
    t_next = jnp.sum(m2 * tr, axis=1, keepdims=True)
    u = (s - t_prev) / (t_next - t_prev)
    u = jnp.minimum(u, 1.0)
    keep_neg = jnp.logical_and(ext > 0.0, s < tt0)
    u = jnp.where(keep_neg, u, jnp.maximum(u, 0.0))

    A = m1 + u * (m2 - m1)
    xW = lax.dot_general(x_ref[0], W_ref[...],
                         (((1,), (1,)), ((), ())),
                         preferred_element_type=jnp.float32)
    y = lax.dot_general(A, xW, (((1,), (0,)), ((), ())),
                        preferred_element_type=jnp.float32) + b_ref[0:1, :]
    ytab_ref[0] = y

    tcol = tT_ref[0]
    it = tr + (tcol - tr) * jnp.float32(0.5)
    disc = (it / jnp.float32(0.1)).astype(jnp.int32)
    gidx_ref[0] = bi * NPAD + disc


def _make_table_and_idx(x, t, W, b, params):
    t3 = t.reshape(B, 1, T)
    tT = t.reshape(B, T, 1)
    b2 = b.reshape(1, D)
    p3 = params.reshape(B, 1, 8)
    return pl.pallas_call(
        _tc_body,
        grid=(B,),
        in_specs=[
            pl.BlockSpec((1, 1, T), lambda i: (i, 0, 0)),
            pl.BlockSpec((1, T, 1), lambda i: (i, 0, 0)),
            pl.BlockSpec((1, T, D), lambda i: (i, 0, 0)),
            pl.BlockSpec((D, D), lambda i: (0, 0)),
            pl.BlockSpec((1, D), lambda i: (0, 0)),
            pl.BlockSpec((1, 1, 8), lambda i: (i, 0, 0)),
        ],
        out_specs=[
            pl.BlockSpec((1, NPAD, D), lambda i: (i, 0, 0)),
            pl.BlockSpec((1, T, T), lambda i: (i, 0, 0)),
        ],
        out_shape=[
            jax.ShapeDtypeStruct((B, NPAD, D), jnp.float32),
            jax.ShapeDtypeStruct((B, T, T), jnp.int32),
        ],
    )(t3, tT, x, W, b2, p3)


@functools.partial(
    pl.kernel,
    mesh=plsc.VectorSubcoreMesh(core_axis_name="c", subcore_axis_name="s"),
    out_type=jax.ShapeDtypeStruct((NROWS, D), jnp.float32),
    scratch_types=[
        pltpu.VMEM((CHUNK,), jnp.int32),
        pltpu.VMEM((CHUNK, D), jnp.float32),
        pltpu.SemaphoreType.DMA,
    ],
)
def _sc_gather(ytab_hbm, gidx_hbm, out_hbm, idx_v, rows_v, sem):
    wid = lax.axis_index("s") * NC + lax.axis_index("c")
    wbase = wid * ROWS_PER_W

    def body(c, carry):
        base = wbase + c * CHUNK
        pltpu.sync_copy(gidx_hbm.at[pl.ds(base, CHUNK)], idx_v)
        pltpu.async_copy(ytab_hbm.at[idx_v], rows_v, sem).wait()
        pltpu.sync_copy(rows_v, out_hbm.at[pl.ds(base, CHUNK)])
        return carry

    lax.fori_loop(0, NCHUNK, body, 0)


def kernel(x, t, W, b):
    t = t.astype(jnp.float32)
    x = x.astype(jnp.float32)
    tmax = jnp.max(t)
    a8 = 8.0 * tmax
    a2 = 2.0 * tmax
    f8 = jnp.floor(a8)
    f2 = jnp.floor(a2)
    N = (f8 + f2 + jnp.floor((a8 - f8) + (a2 - f2))).astype(jnp.int32) + 6
    delta = (tmax + 5 * 0.1) / (N - 1).astype(jnp.float32)
    tt0 = jnp.min(t)
    ext = (t[:, 0] == tt0).astype(jnp.float32)
    params = jnp.stack(
        [jnp.full((B,), delta), jnp.full((B,), tt0), ext] + [jnp.zeros((B,))] * 5,
        axis=1,
    ).astype(jnp.float32)

    ytab, gidx = _make_table_and_idx(x, t, W, b, params)
    out = _sc_gather(ytab.reshape(B * NPAD, D), gidx.reshape(NROWS))
    return out.reshape(B, T, T, 1, D)


# speedup vs baseline: 25.9553x; 1.4699x over previous
import functools

import jax
import jax.numpy as jnp
from jax import lax
from jax.experimental import pallas as pl
from jax.experimental.pallas import tpu as pltpu
from jax.experimental.pallas import tpu_sc as plsc

B, T, D = 4, 256, 128
NPAD = 2568
NROWS = B * T * T
NC, NS = 2, 16
NW = NC * NS
ROWS_PER_W = NROWS // NW
CHUNK = 128
NCHUNK = ROWS_PER_W // CHUNK
SUP = 256
GPB = SUP // CHUNK
NSUP = ROWS_PER_W // SUP
OUTER = NSUP // 2


def _tc_body(t_ref, tT_ref, x_ref, W_ref, b_ref, p_ref, ytab_ref, gidx_ref):
    bi = pl.program_id(0)
    delta = p_ref[0, 0, 0]
    tt0 = p_ref[0, 0, 1]
    ext = p_ref[0, 0, 2]

    tr = t_ref[0]
    s = lax.broadcasted_iota(jnp.int32, (NPAD, 1), 0).astype(jnp.float32) * delta

    cmp = (tr <= s).astype(jnp.int32)
    cnt = jnp.sum(cmp, axis=1, keepdims=True)
    cp = jnp.clip(cnt, 1, T - 1)
    kk = lax.broadcasted_iota(jnp.int32, (NPAD, T), 1)
    m1 = (kk == cp - 1).astype(jnp.float32)
    m2 = (kk == cp).astype(jnp.float32)
    t_prev = jnp.sum(m1 * tr, axis=1, keepdims=True)
    t_next = jnp.sum(m2 * tr, axis=1, keepdims=True)
    u = (s - t_prev) / (t_next - t_prev)
    u = jnp.minimum(u, 1.0)
    keep_neg = jnp.logical_and(ext > 0.0, s < tt0)
    u = jnp.where(keep_neg, u, jnp.maximum(u, 0.0))

    A = m1 + u * (m2 - m1)
    xW = lax.dot_general(x_ref[0], W_ref[...],
                         (((1,), (1,)), ((), ())),
                         preferred_element_type=jnp.float32)
    y = lax.dot_general(A, xW, (((1,), (0,)), ((), ())),
                        preferred_element_type=jnp.float32) + b_ref[0:1, :]
    ytab_ref[0] = y

    tcol = tT_ref[0]
    it = tr + (tcol - tr) * jnp.float32(0.5)
    disc = (it / jnp.float32(0.1)).astype(jnp.int32)
    gidx_ref[0] = bi * NPAD + disc


def _make_table_and_idx(x, t, W, b, params):
    t3 = t.reshape(B, 1, T)
    tT = t.reshape(B, T, 1)
    b2 = b.reshape(1, D)
    p3 = params.reshape(B, 1, 8)
    return pl.pallas_call(
        _tc_body,
        grid=(B,),
        in_specs=[
            pl.BlockSpec((1, 1, T), lambda i: (i, 0, 0)),
            pl.BlockSpec((1, T, 1), lambda i: (i, 0, 0)),
            pl.BlockSpec((1, T, D), lambda i: (i, 0, 0)),
            pl.BlockSpec((D, D), lambda i: (0, 0)),
            pl.BlockSpec((1, D), lambda i: (0, 0)),
            pl.BlockSpec((1, 1, 8), lambda i: (i, 0, 0)),
        ],
        out_specs=[
            pl.BlockSpec((1, NPAD, D), lambda i: (i, 0, 0)),
            pl.BlockSpec((1, T, T), lambda i: (i, 0, 0)),
        ],
        out_shape=[
            jax.ShapeDtypeStruct((B, NPAD, D), jnp.float32),
            jax.ShapeDtypeStruct((B, T, T), jnp.int32),
        ],
    )(t3, tT, x, W, b2, p3)


@functools.partial(
    pl.kernel,
    mesh=plsc.VectorSubcoreMesh(core_axis_name="c", subcore_axis_name="s"),
    out_type=jax.ShapeDtypeStruct((NROWS, D), jnp.float32),
    scratch_types=[
        pltpu.VMEM((ROWS_PER_W,), jnp.int32),
        pltpu.VMEM((SUP, D), jnp.float32),
        pltpu.VMEM((SUP, D), jnp.float32),
        pltpu.SemaphoreType.DMA,
        pltpu.SemaphoreType.DMA,
        pltpu.SemaphoreType.DMA,
        pltpu.SemaphoreType.DMA,
    ],
)
def _sc_gather(ytab_hbm, gidx_hbm, out_hbm, idx_all, rows0, rows1,
               gsem0, gsem1, wsem0, wsem1):
    wid = lax.axis_index("s") * NC + lax.axis_index("c")
    wbase = wid * ROWS_PER_W
    rows = (rows0, rows1)
    gsem = (gsem0, gsem1)
    wsem = (wsem0, wsem1)

    pltpu.sync_copy(gidx_hbm.at[pl.ds(wbase, ROWS_PER_W)], idx_all)

    def fire_gathers(s, kb):
        for h in range(GPB):
            pltpu.async_copy(
                ytab_hbm.at[idx_all.at[pl.ds(s * SUP + h * CHUNK, CHUNK)]],
                rows[kb].at[pl.ds(h * CHUNK, CHUNK)],
                gsem[kb])

    def drain(sem, kb):
        pltpu.make_async_copy(ytab_hbm.at[pl.ds(0, SUP)], rows[kb], sem).wait()

    fire_gathers(0, 0)

    def body(o, carry):
        for k in range(2):
            s = o * 2 + k
            drain(gsem[k], k)

            @pl.when(s + 1 < NSUP)
            def _fire_next():
                @pl.when(s >= 1)
                def _wait_buf():
                    drain(wsem[1 - k], 1 - k)
                fire_gathers(s + 1, 1 - k)

            pltpu.async_copy(rows[k],
                             out_hbm.at[pl.ds(wbase + s * SUP, SUP)],
                             wsem[k])
        return carry

    lax.fori_loop(0, OUTER, body, 0)
    drain(wsem[0], 0)
    drain(wsem[1], 1)


def kernel(x, t, W, b):
    t = t.astype(jnp.float32)
    x = x.astype(jnp.float32)
    tmax = jnp.max(t)
    a8 = 8.0 * tmax
    a2 = 2.0 * tmax
    f8 = jnp.floor(a8)
    f2 = jnp.floor(a2)
    N = (f8 + f2 + jnp.floor((a8 - f8) + (a2 - f2))).astype(jnp.int32) + 6
    delta = (tmax + 5 * 0.1) / (N - 1).astype(jnp.float32)
    tt0 = jnp.min(t)
    ext = (t[:, 0] == tt0).astype(jnp.float32)
    params = jnp.stack(
        [jnp.full((B,), delta), jnp.full((B,), tt0), ext] + [jnp.zeros((B,))] * 5,
        axis=1,
    ).astype(jnp.float32)

    ytab, gidx = _make_table_and_idx(x, t, W, b, params)
    out = _sc_gather(ytab.reshape(B * NPAD, D), gidx.reshape(NROWS))
    return out.reshape(B, T, T, 1, D)


# speedup vs baseline: 26.6631x; 1.0273x over previous
import functools

import jax
import jax.numpy as jnp
from jax import lax
from jax.experimental import pallas as pl
from jax.experimental.pallas import tpu as pltpu
from jax.experimental.pallas import tpu_sc as plsc

B, T, D = 4, 256, 128
NPAD = 2568
NROWS = B * T * T
NC, NS = 2, 16
NW = NC * NS
ROWS_PER_W = NROWS // NW
CHUNK = 128
NCHUNK = ROWS_PER_W // CHUNK
NBUF = 4
OUTER = NCHUNK // NBUF


def _tc_body(t_ref, tT_ref, x_ref, W_ref, b_ref, p_ref, ytab_ref, gidx_ref):
    bi = pl.program_id(0)
    delta = p_ref[0, 0, 0]
    tt0 = p_ref[0, 0, 1]
    ext = p_ref[0, 0, 2]

    tr = t_ref[0]
    s = lax.broadcasted_iota(jnp.int32, (NPAD, 1), 0).astype(jnp.float32) * delta

    cmp = (tr <= s).astype(jnp.int32)
    cnt = jnp.sum(cmp, axis=1, keepdims=True)
    cp = jnp.clip(cnt, 1, T - 1)
    kk = lax.broadcasted_iota(jnp.int32, (NPAD, T), 1)
    m1 = (kk == cp - 1).astype(jnp.float32)
    m2 = (kk == cp).astype(jnp.float32)
    t_prev = jnp.sum(m1 * tr, axis=1, keepdims=True)
    t_next = jnp.sum(m2 * tr, axis=1, keepdims=True)
    u = (s - t_prev) / (t_next - t_prev)
    u = jnp.minimum(u, 1.0)
    keep_neg = jnp.logical_and(ext > 0.0, s < tt0)
    u = jnp.where(keep_neg, u, jnp.maximum(u, 0.0))

    A = m1 + u * (m2 - m1)
    xW = lax.dot_general(x_ref[0], W_ref[...],
                         (((1,), (1,)), ((), ())),
                         preferred_element_type=jnp.float32)
    y = lax.dot_general(A, xW, (((1,), (0,)), ((), ())),
                        preferred_element_type=jnp.float32) + b_ref[0:1, :]
    ytab_ref[0] = y

    tcol = tT_ref[0]
    it = tr + (tcol - tr) * jnp.float32(0.5)
    disc = (it / jnp.float32(0.1)).astype(jnp.int32)
    gidx_ref[0] = bi * NPAD + disc


def _make_table_and_idx(x, t, W, b, params):
    t3 = t.reshape(B, 1, T)
    tT = t.reshape(B, T, 1)
    b2 = b.reshape(1, D)
    p3 = params.reshape(B, 1, 8)
    return pl.pallas_call(
        _tc_body,
        grid=(B,),
        in_specs=[
            pl.BlockSpec((1, 1, T), lambda i: (i, 0, 0)),
            pl.BlockSpec((1, T, 1), lambda i: (i, 0, 0)),
            pl.BlockSpec((1, T, D), lambda i: (i, 0, 0)),
            pl.BlockSpec((D, D), lambda i: (0, 0)),
            pl.BlockSpec((1, D), lambda i: (0, 0)),
            pl.BlockSpec((1, 1, 8), lambda i: (i, 0, 0)),
        ],
        out_specs=[
            pl.BlockSpec((1, NPAD, D), lambda i: (i, 0, 0)),
            pl.BlockSpec((1, T, T), lambda i: (i, 0, 0)),
        ],
        out_shape=[
            jax.ShapeDtypeStruct((B, NPAD, D), jnp.float32),
            jax.ShapeDtypeStruct((B, T, T), jnp.int32),
        ],
    )(t3, tT, x, W, b2, p3)


@functools.partial(
    pl.kernel,
    mesh=plsc.VectorSubcoreMesh(core_axis_name="c", subcore_axis_name="s"),
    out_type=jax.ShapeDtypeStruct((NROWS, D), jnp.float32),
    scratch_types=[
        pltpu.VMEM((ROWS_PER_W,), jnp.int32),
    ] + [pltpu.VMEM((CHUNK, D), jnp.float32)] * NBUF
      + [pltpu.SemaphoreType.DMA] * (2 * NBUF),
)
def _sc_gather(ytab_hbm, gidx_hbm, out_hbm, idx_all, *bufsems):
    rows = bufsems[:NBUF]
    gsem = bufsems[NBUF:2 * NBUF]
    wsem = bufsems[2 * NBUF:]
    wid = lax.axis_index("s") * NC + lax.axis_index("c")
    wbase = wid * ROWS_PER_W

    pltpu.sync_copy(gidx_hbm.at[pl.ds(wbase, ROWS_PER_W)], idx_all)

    def fire_gather(s, kb):
        pltpu.async_copy(
            ytab_hbm.at[idx_all.at[pl.ds(s * CHUNK, CHUNK)]],
            rows[kb], gsem[kb])

    def drain(sem, kb):
        pltpu.make_async_copy(ytab_hbm.at[pl.ds(0, CHUNK)], rows[kb], sem).wait()

    for k in range(NBUF - 1):
        fire_gather(k, k)

    def body(o, carry):
        for k in range(NBUF):
            s = o * NBUF + k
            drain(gsem[k], k)
            pltpu.async_copy(rows[k],
                             out_hbm.at[pl.ds(wbase + s * CHUNK, CHUNK)],
                             wsem[k])
            kn = (k + NBUF - 1) % NBUF

            @pl.when(s + NBUF - 1 < NCHUNK)
            def _fire_next():
                @pl.when(s >= 1)
                def _wait_buf():
                    drain(wsem[kn], kn)
                fire_gather(s + NBUF - 1, kn)
        return carry

    lax.fori_loop(0, OUTER, body, 0)
    for k in range(NBUF):
        drain(wsem[k], k)


def kernel(x, t, W, b):
    t = t.astype(jnp.float32)
    x = x.astype(jnp.float32)
    tmax = jnp.max(t)
    a8 = 8.0 * tmax
    a2 = 2.0 * tmax
    f8 = jnp.floor(a8)
    f2 = jnp.floor(a2)
    N = (f8 + f2 + jnp.floor((a8 - f8) + (a2 - f2))).astype(jnp.int32) + 6
    delta = (tmax + 5 * 0.1) / (N - 1).astype(jnp.float32)
    tt0 = jnp.min(t)
    ext = (t[:, 0] == tt0).astype(jnp.float32)
    params = jnp.stack(
        [jnp.full((B,), delta), jnp.full((B,), tt0), ext] + [jnp.zeros((B,))] * 5,
        axis=1,
    ).astype(jnp.float32)

    ytab, gidx = _make_table_and_idx(x, t, W, b, params)
    out = _sc_gather(ytab.reshape(B * NPAD, D), gidx.reshape(NROWS))
    return out.reshape(B, T, T, 1, D)
